# Initial kernel scaffold; baseline (speedup 1.0000x reference)
import jax
import jax.numpy as jnp
from jax.experimental import pallas as pl


def kernel(query, key, value, Wl, bl):
    raise NotImplementedError("write your pallas kernel here")


# speedup vs baseline: 1.1248x; 1.1248x over previous
import functools

import numpy as np
import jax
import jax.numpy as jnp
from jax import lax
from jax.experimental import pallas as pl
from jax.experimental.pallas import tpu as pltpu
from jax.experimental.pallas import tpu_sc as plsc

BLKQ, BLKK = 128, 64
TOPK_RATIO = 0.5

_pallas_call = pl.pallas_call


def _softmax_last(x):
    m = jnp.max(x, axis=-1, keepdims=True)
    e = jnp.exp(x - m)
    return e / jnp.sum(e, axis=-1, keepdims=True)


def _prep_body(q_ref, k_ref, v_ref, wl_ref, scores_ref, m_ref, ksum_ref):
    q = q_ref[0]
    k = k_ref[0]
    v = v_ref[0]
    L, D = q.shape
    nqb, nkb = L // BLKQ, L // BLKK
    scale = np.float32(1.0 / np.sqrt(D))

    q_blk = jnp.mean(q.reshape(nqb, BLKQ, D), axis=1)
    k_blk = jnp.mean(k.reshape(nkb, BLKK, D), axis=1)
    scores_ref[0] = scale * lax.dot_general(
        q_blk, k_blk, (((1,), (1,)), ((), ())),
        preferred_element_type=jnp.float32)

    kl = _softmax_last(k)
    kv = lax.dot_general(kl, v, (((0,), (0,)), ((), ())),
                         preferred_element_type=jnp.float32)
    m_ref[0] = lax.dot_general(kv, wl_ref[...], (((1,), (1,)), ((), ())),
                               preferred_element_type=jnp.float32)
    ksum_ref[0] = jnp.sum(kl, axis=0, keepdims=True)


def _prep(q, k, v, Wl):
    H, L, D = q.shape
    nqb, nkb = L // BLKQ, L // BLKK
    return _pallas_call(
        _prep_body,
        grid=(H,),
        in_specs=[
            pl.BlockSpec((1, L, D), lambda h: (h, 0, 0)),
            pl.BlockSpec((1, L, D), lambda h: (h, 0, 0)),
            pl.BlockSpec((1, L, D), lambda h: (h, 0, 0)),
            pl.BlockSpec((D, D), lambda h: (0, 0)),
        ],
        out_specs=[
            pl.BlockSpec((1, nqb, nkb), lambda h: (h, 0, 0)),
            pl.BlockSpec((1, D, D), lambda h: (h, 0, 0)),
            pl.BlockSpec((1, 1, D), lambda h: (h, 0, 0)),
        ],
        out_shape=[
            jax.ShapeDtypeStruct((H, nqb, nkb), jnp.float32),
            jax.ShapeDtypeStruct((H, D, D), jnp.float32),
            jax.ShapeDtypeStruct((H, 1, D), jnp.float32),
        ],
    )(q, k, v, Wl)


def _topk_lut(scores2d):
    R = scores2d.shape[0]
    n_workers = 32
    rows_per = R // n_workers
    mesh = plsc.VectorSubcoreMesh(core_axis_name="c", subcore_axis_name="s")

    @functools.partial(
        pl.kernel,
        mesh=mesh,
        out_type=jax.ShapeDtypeStruct((R, 16), jnp.int32),
        scratch_types=[
            pltpu.VMEM((rows_per, 32), jnp.float32),
            pltpu.VMEM((rows_per, 16), jnp.int32),
        ],
    )
    def topk_kernel(s_hbm, lut_hbm, s_v, o_v):
        wid = lax.axis_index("s") * 2 + lax.axis_index("c")
        base = wid * rows_per
        pltpu.sync_copy(s_hbm.at[pl.ds(base, rows_per)], s_v)
        iota = lax.iota(jnp.int32, 16)
        one = jnp.full((16,), 1, jnp.int32)
        zero = jnp.full((16,), 0, jnp.int32)

        def rot(vec, idxv):
            dnums = lax.GatherDimensionNumbers(
                offset_dims=(), collapsed_slice_dims=(0,),
                start_index_map=(0,))
            return lax.gather(
                vec, idxv[:, None], dnums, slice_sizes=(1,),
                mode=lax.GatherScatterMode.PROMISE_IN_BOUNDS)

        def cnt(cond):
            return jnp.where(cond, one, zero)

        for r in range(rows_per):
            s_lo = s_v[r, pl.ds(0, 16)]
            s_hi = s_v[r, pl.ds(16, 16)]
            rank_lo = zero
            rank_hi = zero
            for kk in range(16):
                idxv = jnp.bitwise_and(iota + kk, 15)
                r_lo = rot(s_lo, idxv)
                r_hi = rot(s_hi, idxv)
                rank_lo = (rank_lo + cnt(r_lo > s_lo) + cnt(r_hi > s_lo)
                           + cnt((r_lo == s_lo) & (idxv < iota)))
                rank_hi = (rank_hi + cnt(r_lo > s_hi) + cnt(r_hi > s_hi)
                           + cnt((r_lo == s_hi))
                           + cnt((r_hi == s_hi) & (idxv < iota)))
            out_row = zero
            for kk in range(16):
                idxv = jnp.bitwise_and(iota + kk, 15)
                rl = rot(rank_lo, idxv)
                rh = rot(rank_hi, idxv)
                out_row = (out_row
                           + jnp.where(rl == iota, idxv, zero)
                           + jnp.where(rh == iota, idxv + 16, zero))
            o_v[r, pl.ds(0, 16)] = out_row
        pltpu.sync_copy(o_v, lut_hbm.at[pl.ds(base, rows_per)])

    return topk_kernel(scores2d)


def _flash_body(lut_ref, q_ref, k_ref, v_ref, m_ref, ksum_ref, bl_ref,
                o_ref, *, nqb, topk):
    h = pl.program_id(0)
    qb = pl.program_id(1)
    q = q_ref[0]
    D = q.shape[-1]
    scale = np.float32(1.0 / np.sqrt(D))
    base = (h * nqb + qb) * topk

    m_i = jnp.full((BLKQ, 1), -1e30, jnp.float32)
    l_i = jnp.zeros((BLKQ, 1), jnp.float32)
    acc = jnp.zeros((BLKQ, D), jnp.float32)
    for t in range(topk):
        idx = lut_ref[base + t]
        kt = k_ref[0, pl.ds(idx * BLKK, BLKK), :]
        vt = v_ref[0, pl.ds(idx * BLKK, BLKK), :]
        s = scale * lax.dot_general(q, kt, (((1,), (1,)), ((), ())),
                                    preferred_element_type=jnp.float32)
        m_new = jnp.maximum(m_i, jnp.max(s, axis=-1, keepdims=True))
        alpha = jnp.exp(m_i - m_new)
        p = jnp.exp(s - m_new)
        l_i = l_i * alpha + jnp.sum(p, axis=-1, keepdims=True)
        acc = acc * alpha + lax.dot_general(p, vt, (((1,), (0,)), ((), ())),
                                            preferred_element_type=jnp.float32)
        m_i = m_new
    o_s = acc / l_i

    ql = _softmax_last(q)
    denom = 1e-5 + jnp.sum(ql * ksum_ref[0], axis=-1, keepdims=True)
    o_l = lax.dot_general(ql, m_ref[0], (((1,), (0,)), ((), ())),
                          preferred_element_type=jnp.float32) / denom
    o_ref[0] = o_s + o_l + bl_ref[...]


def _flash(lut_flat, q, k, v, M, ksum, bl2):
    H, L, D = q.shape
    nqb, topk = L // BLKQ, lut_flat.shape[0] // (H * (L // BLKQ))
    grid_spec = pltpu.PrefetchScalarGridSpec(
        num_scalar_prefetch=1,
        grid=(H, nqb),
        in_specs=[
            pl.BlockSpec((1, BLKQ, D), lambda h, qb, lut: (h, qb, 0)),
            pl.BlockSpec((1, L, D), lambda h, qb, lut: (h, 0, 0)),
            pl.BlockSpec((1, L, D), lambda h, qb, lut: (h, 0, 0)),
            pl.BlockSpec((1, D, D), lambda h, qb, lut: (h, 0, 0)),
            pl.BlockSpec((1, 1, D), lambda h, qb, lut: (h, 0, 0)),
            pl.BlockSpec((1, D), lambda h, qb, lut: (0, 0)),
        ],
        out_specs=pl.BlockSpec((1, BLKQ, D), lambda h, qb, lut: (h, qb, 0)),
    )
    body = functools.partial(_flash_body, nqb=nqb, topk=topk)
    return _pallas_call(
        body,
        grid_spec=grid_spec,
        out_shape=jax.ShapeDtypeStruct((H, L, D), jnp.float32),
    )(lut_flat, q, k, v, M, ksum, bl2)


def kernel(query, key, value, Wl, bl):
    B, L, H, D = query.shape
    q = jnp.transpose(query[0], (1, 0, 2))
    k = jnp.transpose(key[0], (1, 0, 2))
    v = jnp.transpose(value[0], (1, 0, 2))

    scores, M, ksum = _prep(q, k, v, Wl)
    nqb, nkb = L // BLKQ, L // BLKK
    lut = _topk_lut(scores.reshape(H * nqb, nkb))
    out = _flash(lut.reshape(-1), q, k, v, M, ksum, bl.reshape(1, D))
    return jnp.transpose(out, (1, 0, 2))[None]


# speedup vs baseline: 1.4282x; 1.2698x over previous
import functools

import numpy as np
import jax
import jax.numpy as jnp
from jax import lax
from jax.experimental import pallas as pl
from jax.experimental.pallas import tpu as pltpu
from jax.experimental.pallas import tpu_sc as plsc

BLKQ, BLKK = 128, 64
TOPK_RATIO = 0.5

_pallas_call = pl.pallas_call


def _softmax_last(x):
    m = jnp.max(x, axis=-1, keepdims=True)
    e = jnp.exp(x - m)
    return e / jnp.sum(e, axis=-1, keepdims=True)


def _prep_body(q_ref, k_ref, v_ref, wl_ref, scores_ref, m_ref, ksum_ref):
    q = q_ref[0]
    k = k_ref[0]
    v = v_ref[0]
    L, D = q.shape
    nqb, nkb = L // BLKQ, L // BLKK
    scale = np.float32(1.0 / np.sqrt(D))

    q_blk = jnp.mean(q.reshape(nqb, BLKQ, D), axis=1)
    k_blk = jnp.mean(k.reshape(nkb, BLKK, D), axis=1)
    scores_ref[0] = scale * lax.dot_general(
        q_blk, k_blk, (((1,), (1,)), ((), ())),
        preferred_element_type=jnp.float32)

    kl = _softmax_last(k)
    kv = lax.dot_general(kl, v, (((0,), (0,)), ((), ())),
                         preferred_element_type=jnp.float32)
    m_ref[0] = lax.dot_general(kv, wl_ref[...], (((1,), (1,)), ((), ())),
                               preferred_element_type=jnp.float32)
    ksum_ref[0] = jnp.sum(kl, axis=0, keepdims=True)


def _prep(q, k, v, Wl):
    H, L, D = q.shape
    nqb, nkb = L // BLKQ, L // BLKK
    return _pallas_call(
        _prep_body,
        grid=(H,),
        in_specs=[
            pl.BlockSpec((1, L, D), lambda h: (h, 0, 0)),
            pl.BlockSpec((1, L, D), lambda h: (h, 0, 0)),
            pl.BlockSpec((1, L, D), lambda h: (h, 0, 0)),
            pl.BlockSpec((D, D), lambda h: (0, 0)),
        ],
        out_specs=[
            pl.BlockSpec((1, nqb, nkb), lambda h: (h, 0, 0)),
            pl.BlockSpec((1, D, D), lambda h: (h, 0, 0)),
            pl.BlockSpec((1, 1, D), lambda h: (h, 0, 0)),
        ],
        out_shape=[
            jax.ShapeDtypeStruct((H, nqb, nkb), jnp.float32),
            jax.ShapeDtypeStruct((H, D, D), jnp.float32),
            jax.ShapeDtypeStruct((H, 1, D), jnp.float32),
        ],
    )(q, k, v, Wl)


def _topk_lut(scores2d):
    R = scores2d.shape[0]
    n_workers = 32
    rows_per = R // n_workers
    mesh = plsc.VectorSubcoreMesh(core_axis_name="c", subcore_axis_name="s")

    @functools.partial(
        pl.kernel,
        mesh=mesh,
        out_type=jax.ShapeDtypeStruct((R, 16), jnp.int32),
        scratch_types=[
            pltpu.VMEM((rows_per, 32), jnp.float32),
            pltpu.VMEM((rows_per, 16), jnp.int32),
        ],
    )
    def topk_kernel(s_hbm, lut_hbm, s_v, o_v):
        wid = lax.axis_index("s") * 2 + lax.axis_index("c")
        base = wid * rows_per
        pltpu.sync_copy(s_hbm.at[pl.ds(base, rows_per)], s_v)
        iota = lax.iota(jnp.int32, 16)
        one = jnp.full((16,), 1, jnp.int32)
        zero = jnp.full((16,), 0, jnp.int32)

        def rot(vec, idxv):
            dnums = lax.GatherDimensionNumbers(
                offset_dims=(), collapsed_slice_dims=(0,),
                start_index_map=(0,))
            return lax.gather(
                vec, idxv[:, None], dnums, slice_sizes=(1,),
                mode=lax.GatherScatterMode.PROMISE_IN_BOUNDS)

        def cnt(cond):
            return jnp.where(cond, one, zero)

        for r in range(rows_per):
            s_lo = s_v[r, pl.ds(0, 16)]
            s_hi = s_v[r, pl.ds(16, 16)]
            rank_lo = zero
            rank_hi = zero
            for kk in range(16):
                idxv = jnp.bitwise_and(iota + kk, 15)
                r_lo = rot(s_lo, idxv)
                r_hi = rot(s_hi, idxv)
                rank_lo = (rank_lo + cnt(r_lo > s_lo) + cnt(r_hi > s_lo)
                           + cnt((r_lo == s_lo) & (idxv < iota)))
                rank_hi = (rank_hi + cnt(r_lo > s_hi) + cnt(r_hi > s_hi)
                           + cnt((r_lo == s_hi))
                           + cnt((r_hi == s_hi) & (idxv < iota)))
            out_row = zero
            for kk in range(16):
                idxv = jnp.bitwise_and(iota + kk, 15)
                rl = rot(rank_lo, idxv)
                rh = rot(rank_hi, idxv)
                out_row = (out_row
                           + jnp.where(rl == iota, idxv, zero)
                           + jnp.where(rh == iota, idxv + 16, zero))
            o_v[r, pl.ds(0, 16)] = out_row
        pltpu.sync_copy(o_v, lut_hbm.at[pl.ds(base, rows_per)])

    return topk_kernel(scores2d)


def _flash_body(lut_ref, q_ref, k_ref, v_ref, m_ref, ksum_ref, bl_ref,
                o_ref, *, nqb, topk):
    h = pl.program_id(0)
    qb = pl.program_id(1)
    q = q_ref[0]
    D = q.shape[-1]
    scale = np.float32(1.0 / np.sqrt(D))
    base = (h * nqb + qb) * topk

    chunk = 4
    s_chunks = []
    v_chunks = []
    for c in range(topk // chunk):
        ks = [k_ref[0, pl.ds(lut_ref[base + c * chunk + j] * BLKK, BLKK), :]
              for j in range(chunk)]
        vs = [v_ref[0, pl.ds(lut_ref[base + c * chunk + j] * BLKK, BLKK), :]
              for j in range(chunk)]
        k_c = jnp.concatenate(ks, axis=0)
        v_c = jnp.concatenate(vs, axis=0)
        s_c = scale * lax.dot_general(q, k_c, (((1,), (1,)), ((), ())),
                                      preferred_element_type=jnp.float32)
        s_chunks.append(s_c)
        v_chunks.append(v_c)

    m_i = s_chunks[0].max(axis=-1, keepdims=True)
    for s_c in s_chunks[1:]:
        m_i = jnp.maximum(m_i, s_c.max(axis=-1, keepdims=True))
    l_i = jnp.zeros((BLKQ, 1), jnp.float32)
    acc = jnp.zeros((BLKQ, D), jnp.float32)
    for s_c, v_c in zip(s_chunks, v_chunks):
        p = jnp.exp(s_c - m_i)
        l_i = l_i + jnp.sum(p, axis=-1, keepdims=True)
        acc = acc + lax.dot_general(p, v_c, (((1,), (0,)), ((), ())),
                                    preferred_element_type=jnp.float32)
    o_s = acc / l_i

    ql = _softmax_last(q)
    denom = 1e-5 + jnp.sum(ql * ksum_ref[0], axis=-1, keepdims=True)
    o_l = lax.dot_general(ql, m_ref[0], (((1,), (0,)), ((), ())),
                          preferred_element_type=jnp.float32) / denom
    o_ref[0] = o_s + o_l + bl_ref[...]


def _flash(lut_flat, q, k, v, M, ksum, bl2):
    H, L, D = q.shape
    nqb, topk = L // BLKQ, lut_flat.shape[0] // (H * (L // BLKQ))
    grid_spec = pltpu.PrefetchScalarGridSpec(
        num_scalar_prefetch=1,
        grid=(H, nqb),
        in_specs=[
            pl.BlockSpec((1, BLKQ, D), lambda h, qb, lut: (h, qb, 0)),
            pl.BlockSpec((1, L, D), lambda h, qb, lut: (h, 0, 0)),
            pl.BlockSpec((1, L, D), lambda h, qb, lut: (h, 0, 0)),
            pl.BlockSpec((1, D, D), lambda h, qb, lut: (h, 0, 0)),
            pl.BlockSpec((1, 1, D), lambda h, qb, lut: (h, 0, 0)),
            pl.BlockSpec((1, D), lambda h, qb, lut: (0, 0)),
        ],
        out_specs=pl.BlockSpec((1, BLKQ, D), lambda h, qb, lut: (h, qb, 0)),
    )
    body = functools.partial(_flash_body, nqb=nqb, topk=topk)
    return _pallas_call(
        body,
        grid_spec=grid_spec,
        out_shape=jax.ShapeDtypeStruct((H, L, D), jnp.float32),
    )(lut_flat, q, k, v, M, ksum, bl2)


def kernel(query, key, value, Wl, bl):
    B, L, H, D = query.shape
    q = jnp.transpose(query[0], (1, 0, 2))
    k = jnp.transpose(key[0], (1, 0, 2))
    v = jnp.transpose(value[0], (1, 0, 2))

    scores, M, ksum = _prep(q, k, v, Wl)
    nqb, nkb = L // BLKQ, L // BLKK
    lut = _topk_lut(scores.reshape(H * nqb, nkb))
    out = _flash(lut.reshape(-1), q, k, v, M, ksum, bl.reshape(1, D))
    return jnp.transpose(out, (1, 0, 2))[None]
